# baseline (speedup 1.0000x reference)
import functools

import jax
import jax.numpy as jnp
from jax import lax
from jax.experimental import pallas as pl
from jax.experimental.pallas import tpu as pltpu
from jax.experimental.pallas import tpu_sc as plsc

NBUF = 4
P = 4


def _make_sc_kernel(B, S, D, NC, NS, L):
    NW = NC * NS
    BW = B // NW
    R = BW * P
    n_chunks = S // P
    n_super = n_chunks // NBUF
    nj = D // L
    mesh = plsc.VectorSubcoreMesh(core_axis_name="c", subcore_axis_name="s")

    @functools.partial(
        pl.kernel,
        mesh=mesh,
        out_type=jax.ShapeDtypeStruct((B * S, D), jnp.float32),
        scratch_types=(
            [pltpu.VMEM((S // P, R), jnp.int32),
             pltpu.VMEM((S, D), jnp.float32),
             pltpu.VMEM((NBUF, R), jnp.int32)]
            + [pltpu.VMEM((R, D), jnp.float32) for _ in range(NBUF)]
            + [pltpu.SemaphoreType.DMA for _ in range(2 * NBUF + 1)]
        ),
    )
    def k(idx_hbm, tok_hbm, pos_hbm, out_hbm, idx_slab, pos_v, oidx, *rest):
        rows_vs = rest[:NBUF]
        gsems = rest[NBUF:2 * NBUF]
        wsems = rest[2 * NBUF:3 * NBUF]
        psem = rest[3 * NBUF]
        wid = lax.axis_index("s") * NC + lax.axis_index("c")
        b0 = wid * BW
        pltpu.async_copy(pos_hbm, pos_v, psem)
        pltpu.sync_copy(idx_hbm.at[wid], idx_slab)

        iota = jnp.arange(0, 16, dtype=jnp.int32)
        pbits = P.bit_length() - 1
        obase = []
        for j in range(R // 16):
            r = iota + (16 * j)
            obase.append((b0 + lax.shift_right_logical(r, pbits)) * S
                         + jnp.bitwise_and(r, P - 1))

        def start_gather(c, b):
            s0 = c * P
            for j in range(R // 16):
                oidx[b, pl.ds(16 * j, 16)] = obase[j] + s0
            pltpu.async_copy(tok_hbm.at[idx_slab.at[c]], rows_vs[b], gsems[b])

        def wait_gather(c, b):
            pltpu.make_async_copy(
                tok_hbm.at[idx_slab.at[c]], rows_vs[b], gsems[b]).wait()

        def wait_wb(b):
            pltpu.make_async_copy(
                rows_vs[b], out_hbm.at[oidx.at[b]], wsems[b]).wait()

        def consume(c, b):
            wait_gather(c, b)
            rv = rows_vs[b]
            for p in range(P):
                s = c * P + p
                pv = [pos_v[s, pl.ds(L * j, L)] for j in range(nj)]

                def row_body(i, _):
                    r = i * P + p
                    for j in range(nj):
                        sl = pl.ds(L * j, L)
                        rv[r, sl] = rv[r, sl] + pv[j]
                    return 0

                lax.fori_loop(0, BW, row_body, 0, unroll=2)
            pltpu.async_copy(rv, out_hbm.at[oidx.at[b]], wsems[b])

        start_gather(0, 0)
        start_gather(1, 1)
        pltpu.make_async_copy(pos_hbm, pos_v, psem).wait()

        for b in range(NBUF):
            b2 = (b + 2) % NBUF
            if b >= 2:
                wait_wb(b2)
            start_gather(b + 2, b2)
            consume(b, b)

        def super_body(sg, _):
            for b in range(NBUF):
                c = sg * NBUF + b
                b2 = (b + 2) % NBUF
                wait_wb(b2)
                start_gather(c + 2, b2)
                consume(c, b)
            return 0

        tail = n_chunks % NBUF
        if tail == 0:
            lax.fori_loop(1, n_super - 1, super_body, 0)
            for b in range(NBUF):
                c = (n_super - 1) * NBUF + b
                b2 = (b + 2) % NBUF
                if b < 2:
                    wait_wb(b2)
                    start_gather(c + 2, b2)
                consume(c, b)
        else:
            lax.fori_loop(1, n_super, super_body, 0)
            for t in range(tail):
                consume(n_super * NBUF + t, t)

        for b in range(NBUF):
            wait_wb(b)

    return k


def kernel(inputs, token_table, pos_table):
    B, S = inputs.shape
    V, D = token_table.shape
    info = plsc.get_sparse_core_info()
    NC, NS, L = info.num_cores, info.num_subcores, info.num_lanes
    NW = NC * NS
    BW = B // NW
    idx_t = (inputs.astype(jnp.int32).reshape(NW, BW, S // P, P)
             .transpose(0, 2, 1, 3).reshape(NW, S // P, BW * P))
    k = _make_sc_kernel(B, S, D, NC, NS, L)
    out = k(idx_t, token_table, pos_table)
    return out.reshape(B, S, D)


# speedup vs baseline: 1.0156x; 1.0156x over previous
import functools

import jax
import jax.numpy as jnp
from jax import lax
from jax.experimental import pallas as pl
from jax.experimental.pallas import tpu as pltpu
from jax.experimental.pallas import tpu_sc as plsc

NBUF = 5
P = 4


def _make_sc_kernel(B, S, D, NC, NS, L):
    NW = NC * NS
    BW = B // NW
    R = BW * P
    n_chunks = S // P
    n_super = n_chunks // NBUF
    nj = D // L
    mesh = plsc.VectorSubcoreMesh(core_axis_name="c", subcore_axis_name="s")

    @functools.partial(
        pl.kernel,
        mesh=mesh,
        out_type=jax.ShapeDtypeStruct((B * S, D), jnp.float32),
        scratch_types=(
            [pltpu.VMEM((S // P, R), jnp.int32),
             pltpu.VMEM((S, D), jnp.float32),
             pltpu.VMEM((NBUF, R), jnp.int32)]
            + [pltpu.VMEM((R, D), jnp.float32) for _ in range(NBUF)]
            + [pltpu.SemaphoreType.DMA for _ in range(2 * NBUF + 1)]
        ),
    )
    def k(idx_hbm, tok_hbm, pos_hbm, out_hbm, idx_slab, pos_v, oidx, *rest):
        rows_vs = rest[:NBUF]
        gsems = rest[NBUF:2 * NBUF]
        wsems = rest[2 * NBUF:3 * NBUF]
        psem = rest[3 * NBUF]
        wid = lax.axis_index("s") * NC + lax.axis_index("c")
        b0 = wid * BW
        pltpu.async_copy(pos_hbm, pos_v, psem)
        pltpu.sync_copy(idx_hbm.at[wid], idx_slab)

        iotas = [jnp.arange(16 * j, 16 * (j + 1), dtype=jnp.int32)
                 for j in range(BW // 16)]
        obase = [(b0 + it) * S for it in iotas]

        def start_gather(c, b):
            s0 = c * P
            for p in range(P):
                s = s0 + p
                for j in range(BW // 16):
                    sl = pl.ds(p * BW + 16 * j, 16)
                    oidx[b, sl] = obase[j] + s
            pltpu.async_copy(tok_hbm.at[idx_slab.at[c]], rows_vs[b], gsems[b])

        def wait_gather(c, b):
            pltpu.make_async_copy(
                tok_hbm.at[idx_slab.at[c]], rows_vs[b], gsems[b]).wait()

        def wait_wb(b):
            pltpu.make_async_copy(
                rows_vs[b], out_hbm.at[oidx.at[b]], wsems[b]).wait()

        def consume(c, b):
            wait_gather(c, b)
            rv = rows_vs[b]
            for p in range(P):
                s = c * P + p
                pv = [pos_v[s, pl.ds(L * j, L)] for j in range(nj)]

                def row_body(i, _):
                    r = p * BW + i
                    for j in range(nj):
                        sl = pl.ds(L * j, L)
                        rv[r, sl] = rv[r, sl] + pv[j]
                    return 0

                lax.fori_loop(0, BW, row_body, 0, unroll=2)
            pltpu.async_copy(rv, out_hbm.at[oidx.at[b]], wsems[b])

        start_gather(0, 0)
        start_gather(1, 1)
        pltpu.make_async_copy(pos_hbm, pos_v, psem).wait()

        for b in range(NBUF):
            b2 = (b + 2) % NBUF
            if b >= NBUF - 2:
                wait_wb(b2)
            start_gather(b + 2, b2)
            consume(b, b)

        def super_body(sg, _):
            for b in range(NBUF):
                c = sg * NBUF + b
                b2 = (b + 2) % NBUF
                wait_wb(b2)
                start_gather(c + 2, b2)
                consume(c, b)
            return 0

        tail = n_chunks % NBUF
        if tail == 0:
            lax.fori_loop(1, n_super - 1, super_body, 0)
            for b in range(NBUF):
                c = (n_super - 1) * NBUF + b
                b2 = (b + 2) % NBUF
                if b < NBUF - 2:
                    wait_wb(b2)
                    start_gather(c + 2, b2)
                consume(c, b)
        else:
            lax.fori_loop(1, n_super, super_body, 0)
            for t in range(tail):
                consume(n_super * NBUF + t, t)

        for b in range(NBUF):
            wait_wb(b)

    return k


def kernel(inputs, token_table, pos_table):
    B, S = inputs.shape
    V, D = token_table.shape
    info = plsc.get_sparse_core_info()
    NC, NS, L = info.num_cores, info.num_subcores, info.num_lanes
    NW = NC * NS
    BW = B // NW
    idx_t = (inputs.astype(jnp.int32).reshape(NW, BW, S).transpose(0, 2, 1)
             .reshape(NW, S // P, P * BW))
    k = _make_sc_kernel(B, S, D, NC, NS, L)
    out = k(idx_t, token_table, pos_table)
    return out.reshape(B, S, D)


# speedup vs baseline: 1.0310x; 1.0152x over previous
import functools

import jax
import jax.numpy as jnp
from jax import lax
from jax.experimental import pallas as pl
from jax.experimental.pallas import tpu as pltpu
from jax.experimental.pallas import tpu_sc as plsc

NBUF = 4
P = 4


def _make_sc_kernel(B, S, D, NC, NS, L):
    NW = NC * NS
    BW = B // NW
    R = BW * P
    n_chunks = S // P
    n_super = n_chunks // NBUF
    nj = D // L
    mesh = plsc.VectorSubcoreMesh(core_axis_name="c", subcore_axis_name="s")

    @functools.partial(
        pl.kernel,
        mesh=mesh,
        out_type=jax.ShapeDtypeStruct((B * S, D), jnp.float32),
        scratch_types=(
            [pltpu.VMEM((S // P, R), jnp.int32),
             pltpu.VMEM((S, D), jnp.float32),
             pltpu.VMEM((NBUF, R), jnp.int32)]
            + [pltpu.VMEM((R, D), jnp.float32) for _ in range(NBUF)]
            + [pltpu.SemaphoreType.DMA for _ in range(2 * NBUF + 1)]
        ),
    )
    def k(idx_hbm, tok_hbm, pos_hbm, out_hbm, idx_slab, pos_v, oidx, *rest):
        rows_vs = rest[:NBUF]
        gsems = rest[NBUF:2 * NBUF]
        wsems = rest[2 * NBUF:3 * NBUF]
        psem = rest[3 * NBUF]
        wid = lax.axis_index("s") * NC + lax.axis_index("c")
        b0 = wid * BW
        pltpu.async_copy(pos_hbm, pos_v, psem)
        pltpu.sync_copy(idx_hbm.at[wid], idx_slab)

        iotas = [jnp.arange(16 * j, 16 * (j + 1), dtype=jnp.int32)
                 for j in range(BW // 16)]
        obase = [(b0 + it) * S for it in iotas]

        def start_gather(c, b):
            s0 = c * P
            for p in range(P):
                s = s0 + p
                for j in range(BW // 16):
                    sl = pl.ds(p * BW + 16 * j, 16)
                    oidx[b, sl] = obase[j] + s
            pltpu.async_copy(tok_hbm.at[idx_slab.at[c]], rows_vs[b], gsems[b])

        def wait_gather(c, b):
            pltpu.make_async_copy(
                tok_hbm.at[idx_slab.at[c]], rows_vs[b], gsems[b]).wait()

        def wait_wb(b):
            pltpu.make_async_copy(
                rows_vs[b], out_hbm.at[oidx.at[b]], wsems[b]).wait()

        def consume(c, b):
            wait_gather(c, b)
            rv = rows_vs[b]
            for p in range(P):
                s = c * P + p
                pv = [pos_v[s, pl.ds(L * j, L)] for j in range(nj)]

                def row_body(i, _):
                    r = p * BW + i
                    for j in range(nj):
                        sl = pl.ds(L * j, L)
                        rv[r, sl] = rv[r, sl] + pv[j]
                    return 0

                lax.fori_loop(0, BW, row_body, 0, unroll=2)
            pltpu.async_copy(rv, out_hbm.at[oidx.at[b]], wsems[b])

        start_gather(0, 0)
        start_gather(1, 1)
        pltpu.make_async_copy(pos_hbm, pos_v, psem).wait()

        for b in range(NBUF):
            b2 = (b + 2) % NBUF
            if b >= NBUF - 2:
                wait_wb(b2)
            start_gather(b + 2, b2)
            consume(b, b)

        def super_body(sg, _):
            for b in range(NBUF):
                c = sg * NBUF + b
                b2 = (b + 2) % NBUF
                wait_wb(b2)
                start_gather(c + 2, b2)
                consume(c, b)
            return 0

        tail = n_chunks % NBUF
        if tail == 0:
            lax.fori_loop(1, n_super - 1, super_body, 0)
            for b in range(NBUF):
                c = (n_super - 1) * NBUF + b
                b2 = (b + 2) % NBUF
                if b < NBUF - 2:
                    wait_wb(b2)
                    start_gather(c + 2, b2)
                consume(c, b)
        else:
            lax.fori_loop(1, n_super, super_body, 0)
            for t in range(tail):
                consume(n_super * NBUF + t, t)

        for b in range(NBUF):
            wait_wb(b)

    return k


def kernel(inputs, token_table, pos_table):
    B, S = inputs.shape
    V, D = token_table.shape
    info = plsc.get_sparse_core_info()
    NC, NS, L = info.num_cores, info.num_subcores, info.num_lanes
    NW = NC * NS
    BW = B // NW
    idx_t = (inputs.astype(jnp.int32).reshape(NW, BW, S).transpose(0, 2, 1)
             .reshape(NW, S // P, P * BW))
    k = _make_sc_kernel(B, S, D, NC, NS, L)
    out = k(idx_t, token_table, pos_table)
    return out.reshape(B, S, D)
